# baseline (speedup 1.0000x reference)
import jax
import jax.numpy as jnp
from jax.experimental import pallas as pl
from jax.experimental.pallas import tpu as pltpu

C = 256
Q = 4
NF = 2048
NN = 1024
BF = 512
BN = 256
STEPS = 16


def _elu(x):
    return jnp.where(x > 0, x, jnp.exp(x) - 1.0)


def _dot(a, b):
    return jax.lax.dot_general(
        a, b, (((1,), (0,)), ((), ())),
        precision=jax.lax.Precision.DEFAULT,
        preferred_element_type=jnp.float32)


def _mm1_kernel(xr_ref, di_ref, f_ref,
                xe_ref, o_ref, fin_ref, xm_ref, om_ref, fm_ref,
                xe_scr):
    i = pl.program_id(0)

    @pl.when(i == 0)
    def _():
        xe = _elu(xr_ref[...])
        xe_scr[...] = xe
        xe_ref[...] = xe
        x3 = xe.reshape(NN, Q, 64)
        xm_ref[0] = jnp.sum(x3, axis=0)
        xm_ref[1] = jnp.sum(x3 * x3, axis=0)

    o = _dot(di_ref[...], xe_scr[...])
    o_ref[...] = o
    o3 = o.reshape(BF // Q, Q, 64)
    fin = _elu(f_ref[...])
    fin_ref[...] = fin

    @pl.when(i == 0)
    def _():
        om_ref[0] = jnp.sum(o3, axis=0)
        om_ref[1] = jnp.sum(o3 * o3, axis=0)
        fm_ref[0] = jnp.sum(fin, axis=0, keepdims=True)
        fm_ref[1] = jnp.sum(fin * fin, axis=0, keepdims=True)

    @pl.when(i > 0)
    def _():
        om_ref[0] += jnp.sum(o3, axis=0)
        om_ref[1] += jnp.sum(o3 * o3, axis=0)
        fm_ref[0] += jnp.sum(fin, axis=0, keepdims=True)
        fm_ref[1] += jnp.sum(fin * fin, axis=0, keepdims=True)


def _mm2_kernel(da_ref, y_ref, o_ref, om_ref):
    i = pl.program_id(0)
    o = _dot(da_ref[...], y_ref[...])
    o_ref[...] = o
    o3 = o.reshape(BN // Q, Q, 64)

    @pl.when(i == 0)
    def _():
        om_ref[0] = jnp.sum(o3, axis=0)
        om_ref[1] = jnp.sum(o3 * o3, axis=0)

    @pl.when(i > 0)
    def _():
        om_ref[0] += jnp.sum(o3, axis=0)
        om_ref[1] += jnp.sum(o3 * o3, axis=0)


def _lin_kernel(a_ref, b_ref, am_ref, bm_ref, wt_ref, g_ref, be_ref,
                bias_ref, o_ref, y_ref, *, n_rows, emit_y):
    inv_n = 1.0 / n_rows
    mean_a = am_ref[0] * inv_n
    var_a = am_ref[1] * inv_n - mean_a * mean_a
    s_a = g_ref[:, :C] * jax.lax.rsqrt(var_a + 1e-5)
    mean_b = bm_ref[0] * inv_n
    var_b = bm_ref[1] * inv_n - mean_b * mean_b
    s_b = g_ref[:, C:] * jax.lax.rsqrt(var_b + 1e-5)
    wa = wt_ref[:C, :]
    wb = wt_ref[C:, :]
    o = _dot(a_ref[...] * s_a, wa) + _dot(b_ref[...] * s_b, wb)
    cvec = (bias_ref[...]
            + _dot(be_ref[:, :C] - mean_a * s_a, wa)
            + _dot(be_ref[:, C:] - mean_b * s_b, wb))
    o = o + cvec
    if emit_y:
        o_ref[...] = o
        y_ref[...] = _elu(o)
    else:
        o_ref[...] = y_ref[...] + o


def kernel(Di, DiA, v, f, g0, be0, W0, b0, g1, be1, W1, b1):
    Di2 = Di.reshape(Q * NF, Q * NN)
    DiA2 = DiA.reshape(Q * NN, Q * NF)
    v2 = v.reshape(NN, C)
    f2 = f.reshape(NF, C)
    xr_raw = v2.reshape(Q * NN, 64)
    W0t = W0.T
    W1t = W1.T
    g0r = g0.reshape(1, 2 * C)
    be0r = be0.reshape(1, 2 * C)
    g1r = g1.reshape(1, 2 * C)
    be1r = be1.reshape(1, 2 * C)
    b0r = b0.reshape(1, C)
    b1r = b1.reshape(1, C)

    zero = lambda i: (0, 0)
    zero3 = lambda i: (0, 0, 0)
    clamp = lambda i: (jnp.minimum(i, STEPS - 1), 0)

    xe, out1, f_in, xmom, omom, fmom = pl.pallas_call(
        _mm1_kernel,
        grid=(STEPS,),
        in_specs=[pl.BlockSpec((Q * NN, 64), zero),
                  pl.BlockSpec((BF, Q * NN), lambda i: (i, 0)),
                  pl.BlockSpec((BF // Q, C), lambda i: (i, 0))],
        out_specs=(pl.BlockSpec((Q * NN, 64), zero),
                   pl.BlockSpec((BF, 64), lambda i: (i, 0)),
                   pl.BlockSpec((BF // Q, C), lambda i: (i, 0)),
                   pl.BlockSpec((2, Q, 64), zero3),
                   pl.BlockSpec((2, Q, 64), zero3),
                   pl.BlockSpec((2, 1, C), zero3)),
        out_shape=(jax.ShapeDtypeStruct((Q * NN, 64), jnp.float32),
                   jax.ShapeDtypeStruct((Q * NF, 64), jnp.float32),
                   jax.ShapeDtypeStruct((NF, C), jnp.float32),
                   jax.ShapeDtypeStruct((2, Q, 64), jnp.float32),
                   jax.ShapeDtypeStruct((2, Q, 64), jnp.float32),
                   jax.ShapeDtypeStruct((2, 1, C), jnp.float32)),
        scratch_shapes=[pltpu.VMEM((Q * NN, 64), jnp.float32)],
        compiler_params=pltpu.CompilerParams(
            dimension_semantics=("arbitrary",)),
    )(xr_raw, Di2, f2)

    out1r = out1.reshape(NF, C)
    omom2 = omom.reshape(2, 1, C)
    xmom2 = xmom.reshape(2, 1, C)

    def lin_call(a, b, am, bm, wt, g, be, bias, n_rows, emit_y, res=None,
                 bm_rows=256):
        m = a.shape[0]
        row = lambda i: (i, 0)
        import functools
        kern = functools.partial(_lin_kernel, n_rows=n_rows, emit_y=emit_y)
        in_specs = [pl.BlockSpec((bm_rows, C), row),
                    pl.BlockSpec((bm_rows, C), row),
                    pl.BlockSpec((2, 1, C), zero3),
                    pl.BlockSpec((2, 1, C), zero3),
                    pl.BlockSpec((2 * C, C), zero),
                    pl.BlockSpec((1, 2 * C), zero),
                    pl.BlockSpec((1, 2 * C), zero),
                    pl.BlockSpec((1, C), zero)]
        args = [a, b, am, bm, wt, g, be, bias]
        if emit_y:
            out_specs = (pl.BlockSpec((bm_rows, C), row),
                         pl.BlockSpec((bm_rows, C), row))
            out_shape = (jax.ShapeDtypeStruct((m, C), jnp.float32),
                         jax.ShapeDtypeStruct((m, C), jnp.float32))
        else:
            in_specs.append(pl.BlockSpec((bm_rows, C), row))
            args.append(res)
            out_specs = pl.BlockSpec((bm_rows, C), row)
            out_shape = jax.ShapeDtypeStruct((m, C), jnp.float32)
        return pl.pallas_call(
            kern if emit_y else _lin_res_wrap(kern),
            grid=(m // bm_rows,),
            in_specs=in_specs,
            out_specs=out_specs,
            out_shape=out_shape,
            compiler_params=pltpu.CompilerParams(
                dimension_semantics=("arbitrary",)),
        )(*args)

    def _lin_res_wrap(kern):
        def body(a, b, am, bm, wt, g, be, bias, res, o):
            kern(a, b, am, bm, wt, g, be, bias, o, res)
        return body

    f_out, y = lin_call(f_in, out1r, fmom, omom2, W0t, g0r, be0r, b0r,
                        n_rows=NF, emit_y=True)

    yr = y.reshape(Q * NF, 64)
    out2, o2mom = pl.pallas_call(
        _mm2_kernel,
        grid=(STEPS,),
        in_specs=[pl.BlockSpec((BN, Q * NF), lambda i: (i, 0)),
                  pl.BlockSpec((Q * NF, 64), zero)],
        out_specs=(pl.BlockSpec((BN, 64), lambda i: (i, 0)),
                   pl.BlockSpec((2, Q, 64), zero3)),
        out_shape=(jax.ShapeDtypeStruct((Q * NN, 64), jnp.float32),
                   jax.ShapeDtypeStruct((2, Q, 64), jnp.float32)),
        compiler_params=pltpu.CompilerParams(
            dimension_semantics=("arbitrary",)),
    )(DiA2, yr)

    out2r = out2.reshape(NN, C)
    o2mom2 = o2mom.reshape(2, 1, C)
    x_in = xe.reshape(NN, C)

    v_out = lin_call(x_in, out2r, xmom2, o2mom2, W1t, g1r, be1r, b1r,
                     n_rows=NN, emit_y=False, res=v2)

    return (v_out.reshape(v.shape), f_out.reshape(f.shape))


# speedup vs baseline: 1.0178x; 1.0178x over previous
import jax
import jax.numpy as jnp
from jax.experimental import pallas as pl
from jax.experimental.pallas import tpu as pltpu

C = 256
Q = 4
NF = 2048
NN = 1024
BF = 512
BN = 512
STEPS = 16


def _elu(x):
    return jnp.where(x > 0, x, jnp.exp(x) - 1.0)


def _dot(a, b):
    return jax.lax.dot_general(
        a, b, (((1,), (0,)), ((), ())),
        precision=jax.lax.Precision.DEFAULT,
        preferred_element_type=jnp.float32)


def _mm1_kernel(xr_ref, di_ref, f_ref,
                xe_ref, o_ref, fin_ref, xm_ref, om_ref, fm_ref,
                xe_scr):
    i = pl.program_id(0)

    @pl.when(i == 0)
    def _():
        xe = _elu(xr_ref[...])
        xe_scr[...] = xe
        xe_ref[...] = xe
        x3 = xe.reshape(NN, Q, 64)
        xm_ref[0] = jnp.sum(x3, axis=0)
        xm_ref[1] = jnp.sum(x3 * x3, axis=0)

    o = _dot(di_ref[...], xe_scr[...])
    o_ref[...] = o
    o3 = o.reshape(BF // Q, Q, 64)
    fin = _elu(f_ref[...])
    fin_ref[...] = fin

    @pl.when(i == 0)
    def _():
        om_ref[0] = jnp.sum(o3, axis=0)
        om_ref[1] = jnp.sum(o3 * o3, axis=0)
        fm_ref[0] = jnp.sum(fin, axis=0, keepdims=True)
        fm_ref[1] = jnp.sum(fin * fin, axis=0, keepdims=True)

    @pl.when(i > 0)
    def _():
        om_ref[0] += jnp.sum(o3, axis=0)
        om_ref[1] += jnp.sum(o3 * o3, axis=0)
        fm_ref[0] += jnp.sum(fin, axis=0, keepdims=True)
        fm_ref[1] += jnp.sum(fin * fin, axis=0, keepdims=True)


def _mm2_kernel(da_ref, y_ref, o_ref, om_ref):
    i = pl.program_id(0)
    o = _dot(da_ref[...], y_ref[...])
    o_ref[...] = o
    o3 = o.reshape(BN // Q, Q, 64)

    @pl.when(i == 0)
    def _():
        om_ref[0] = jnp.sum(o3, axis=0)
        om_ref[1] = jnp.sum(o3 * o3, axis=0)

    @pl.when(i > 0)
    def _():
        om_ref[0] += jnp.sum(o3, axis=0)
        om_ref[1] += jnp.sum(o3 * o3, axis=0)


def _lin_kernel(a_ref, b_ref, am_ref, bm_ref, wt_ref, g_ref, be_ref,
                bias_ref, o_ref, y_ref, *, n_rows, emit_y):
    inv_n = 1.0 / n_rows
    mean_a = am_ref[0] * inv_n
    var_a = am_ref[1] * inv_n - mean_a * mean_a
    s_a = g_ref[:, :C] * jax.lax.rsqrt(var_a + 1e-5)
    mean_b = bm_ref[0] * inv_n
    var_b = bm_ref[1] * inv_n - mean_b * mean_b
    s_b = g_ref[:, C:] * jax.lax.rsqrt(var_b + 1e-5)
    wa = wt_ref[:C, :]
    wb = wt_ref[C:, :]
    o = _dot(a_ref[...] * s_a, wa) + _dot(b_ref[...] * s_b, wb)
    cvec = (bias_ref[...]
            + _dot(be_ref[:, :C] - mean_a * s_a, wa)
            + _dot(be_ref[:, C:] - mean_b * s_b, wb))
    o = o + cvec
    if emit_y:
        o_ref[...] = o
        y_ref[...] = _elu(o)
    else:
        o_ref[...] = y_ref[...] + o


def kernel(Di, DiA, v, f, g0, be0, W0, b0, g1, be1, W1, b1):
    Di2 = Di.reshape(Q * NF, Q * NN)
    DiA2 = DiA.reshape(Q * NN, Q * NF)
    v2 = v.reshape(NN, C)
    f2 = f.reshape(NF, C)
    xr_raw = v2.reshape(Q * NN, 64)
    W0t = W0.T
    W1t = W1.T
    g0r = g0.reshape(1, 2 * C)
    be0r = be0.reshape(1, 2 * C)
    g1r = g1.reshape(1, 2 * C)
    be1r = be1.reshape(1, 2 * C)
    b0r = b0.reshape(1, C)
    b1r = b1.reshape(1, C)

    zero = lambda i: (0, 0)
    zero3 = lambda i: (0, 0, 0)
    clamp = lambda i: (jnp.minimum(i, STEPS - 1), 0)

    xe, out1, f_in, xmom, omom, fmom = pl.pallas_call(
        _mm1_kernel,
        grid=(STEPS,),
        in_specs=[pl.BlockSpec((Q * NN, 64), zero),
                  pl.BlockSpec((BF, Q * NN), lambda i: (i, 0)),
                  pl.BlockSpec((BF // Q, C), lambda i: (i, 0))],
        out_specs=(pl.BlockSpec((Q * NN, 64), zero),
                   pl.BlockSpec((BF, 64), lambda i: (i, 0)),
                   pl.BlockSpec((BF // Q, C), lambda i: (i, 0)),
                   pl.BlockSpec((2, Q, 64), zero3),
                   pl.BlockSpec((2, Q, 64), zero3),
                   pl.BlockSpec((2, 1, C), zero3)),
        out_shape=(jax.ShapeDtypeStruct((Q * NN, 64), jnp.float32),
                   jax.ShapeDtypeStruct((Q * NF, 64), jnp.float32),
                   jax.ShapeDtypeStruct((NF, C), jnp.float32),
                   jax.ShapeDtypeStruct((2, Q, 64), jnp.float32),
                   jax.ShapeDtypeStruct((2, Q, 64), jnp.float32),
                   jax.ShapeDtypeStruct((2, 1, C), jnp.float32)),
        scratch_shapes=[pltpu.VMEM((Q * NN, 64), jnp.float32)],
        compiler_params=pltpu.CompilerParams(
            dimension_semantics=("arbitrary",)),
    )(xr_raw, Di2, f2)

    out1r = out1.reshape(NF, C)
    omom2 = omom.reshape(2, 1, C)
    xmom2 = xmom.reshape(2, 1, C)

    def lin_call(a, b, am, bm, wt, g, be, bias, n_rows, emit_y, res=None,
                 bm_rows=512):
        m = a.shape[0]
        row = lambda i: (i, 0)
        import functools
        kern = functools.partial(_lin_kernel, n_rows=n_rows, emit_y=emit_y)
        in_specs = [pl.BlockSpec((bm_rows, C), row),
                    pl.BlockSpec((bm_rows, C), row),
                    pl.BlockSpec((2, 1, C), zero3),
                    pl.BlockSpec((2, 1, C), zero3),
                    pl.BlockSpec((2 * C, C), zero),
                    pl.BlockSpec((1, 2 * C), zero),
                    pl.BlockSpec((1, 2 * C), zero),
                    pl.BlockSpec((1, C), zero)]
        args = [a, b, am, bm, wt, g, be, bias]
        if emit_y:
            out_specs = (pl.BlockSpec((bm_rows, C), row),
                         pl.BlockSpec((bm_rows, C), row))
            out_shape = (jax.ShapeDtypeStruct((m, C), jnp.float32),
                         jax.ShapeDtypeStruct((m, C), jnp.float32))
        else:
            in_specs.append(pl.BlockSpec((bm_rows, C), row))
            args.append(res)
            out_specs = pl.BlockSpec((bm_rows, C), row)
            out_shape = jax.ShapeDtypeStruct((m, C), jnp.float32)
        return pl.pallas_call(
            kern if emit_y else _lin_res_wrap(kern),
            grid=(m // bm_rows,),
            in_specs=in_specs,
            out_specs=out_specs,
            out_shape=out_shape,
            compiler_params=pltpu.CompilerParams(
                dimension_semantics=("arbitrary",)),
        )(*args)

    def _lin_res_wrap(kern):
        def body(a, b, am, bm, wt, g, be, bias, res, o):
            kern(a, b, am, bm, wt, g, be, bias, o, res)
        return body

    f_out, y = lin_call(f_in, out1r, fmom, omom2, W0t, g0r, be0r, b0r,
                        n_rows=NF, emit_y=True)

    yr = y.reshape(Q * NF, 64)
    out2, o2mom = pl.pallas_call(
        _mm2_kernel,
        grid=(Q * NN // BN,),
        in_specs=[pl.BlockSpec((BN, Q * NF), lambda i: (i, 0)),
                  pl.BlockSpec((Q * NF, 64), zero)],
        out_specs=(pl.BlockSpec((BN, 64), lambda i: (i, 0)),
                   pl.BlockSpec((2, Q, 64), zero3)),
        out_shape=(jax.ShapeDtypeStruct((Q * NN, 64), jnp.float32),
                   jax.ShapeDtypeStruct((2, Q, 64), jnp.float32)),
        compiler_params=pltpu.CompilerParams(
            dimension_semantics=("arbitrary",)),
    )(DiA2, yr)

    out2r = out2.reshape(NN, C)
    o2mom2 = o2mom.reshape(2, 1, C)
    x_in = xe.reshape(NN, C)

    v_out = lin_call(x_in, out2r, xmom2, o2mom2, W1t, g1r, be1r, b1r,
                     n_rows=NN, emit_y=False, res=v2)

    return (v_out.reshape(v.shape), f_out.reshape(f.shape))


# speedup vs baseline: 1.0763x; 1.0574x over previous
import jax
import jax.numpy as jnp
from jax.experimental import pallas as pl
from jax.experimental.pallas import tpu as pltpu

C = 256
Q = 4
NF = 2048
NN = 1024
BF = 512
BD = 512
S1 = 16
S2 = 8


def _elu(x):
    return jnp.where(x > 0, x, jnp.exp(x) - 1.0)


def _dot(a, b):
    return jax.lax.dot_general(
        a, b, (((1,), (0,)), ((), ())),
        precision=jax.lax.Precision.DEFAULT,
        preferred_element_type=jnp.float32)


def _phase1_kernel(xr_ref, di_ref, f_ref, wt_ref, g0a_ref, g0b_ref,
                   be0a_ref, be0b_ref, b0_ref,
                   xe_ref, fout_ref, y_ref,
                   xe_scr, f_scr, o_scr, fm_s, fm_ss, om_s, om_ss):
    i = pl.program_id(0)

    @pl.when(i == 0)
    def _():
        xe = _elu(xr_ref[...])
        xe_scr[...] = xe
        xe_ref[...] = xe

    @pl.when(i < S1)
    def _():
        o = _dot(di_ref[...], xe_scr[...])
        o3 = o.reshape(BF // Q, Q, 64)
        o_scr[pl.ds(i * (BF // Q), BF // Q)] = o3
        fin = _elu(f_ref[...])
        f_scr[pl.ds(i * (BF // Q), BF // Q), :] = fin

        @pl.when(i == 0)
        def _():
            om_s[...] = jnp.sum(o3, axis=0)
            om_ss[...] = jnp.sum(o3 * o3, axis=0)
            fm_s[...] = jnp.sum(fin, axis=0, keepdims=True)
            fm_ss[...] = jnp.sum(fin * fin, axis=0, keepdims=True)

        @pl.when(i > 0)
        def _():
            om_s[...] += jnp.sum(o3, axis=0)
            om_ss[...] += jnp.sum(o3 * o3, axis=0)
            fm_s[...] += jnp.sum(fin, axis=0, keepdims=True)
            fm_ss[...] += jnp.sum(fin * fin, axis=0, keepdims=True)

    @pl.when(i == S1)
    def _():
        inv_n = 1.0 / NF
        mean_a = fm_s[...] * inv_n
        var_a = fm_ss[...] * inv_n - mean_a * mean_a
        s_a = g0a_ref[...] * jax.lax.rsqrt(var_a + 1e-5)
        mean_b = om_s[...] * inv_n
        var_b = om_ss[...] * inv_n - mean_b * mean_b
        s_b = g0b_ref[...] * jax.lax.rsqrt(var_b + 1e-5)

        wa = wt_ref[:C, :]
        acc = _dot(f_scr[...] * s_a, wa)
        cvec = b0_ref[...] + _dot(be0a_ref[...] - mean_a * s_a, wa)
        shift_b = be0b_ref[...] - mean_b * s_b
        for p in range(Q):
            wp = wt_ref[C + 64 * p:C + 64 * (p + 1), :]
            acc += _dot(o_scr[:, p, :] * s_b[p:p + 1, :], wp)
            cvec += _dot(shift_b[p:p + 1, :], wp)
        fo = acc + cvec
        fout_ref[...] = fo
        y_ref[...] = _elu(fo)


def _phase2_kernel(y_ref, da_ref, xe_ref, v_ref, wt_ref, g1a_ref, g1b_ref,
                   be1a_ref, be1b_ref, b1_ref,
                   vout_ref, xz_scr, o_scr, om_s, om_ss):
    i = pl.program_id(0)

    @pl.when(i == 0)
    def _():
        xz_scr[...] = xe_ref[...].reshape(NN, Q, 64)

    @pl.when(i < S2)
    def _():
        o = _dot(da_ref[...], y_ref[...])
        o3 = o.reshape(BD // Q, Q, 64)
        o_scr[pl.ds(i * (BD // Q), BD // Q)] = o3

        @pl.when(i == 0)
        def _():
            om_s[...] = jnp.sum(o3, axis=0)
            om_ss[...] = jnp.sum(o3 * o3, axis=0)

        @pl.when(i > 0)
        def _():
            om_s[...] += jnp.sum(o3, axis=0)
            om_ss[...] += jnp.sum(o3 * o3, axis=0)

    @pl.when(i == S2)
    def _():
        inv_n = 1.0 / NN
        xz = xz_scr[...]
        xm = jnp.sum(xz, axis=0) * inv_n
        xv = jnp.sum(xz * xz, axis=0) * inv_n - xm * xm
        s_a = g1a_ref[...] * jax.lax.rsqrt(xv + 1e-5)
        mean_b = om_s[...] * inv_n
        var_b = om_ss[...] * inv_n - mean_b * mean_b
        s_b = g1b_ref[...] * jax.lax.rsqrt(var_b + 1e-5)

        acc = v_ref[...] + b1_ref[...]
        shift_a = be1a_ref[...] - xm * s_a
        shift_b = be1b_ref[...] - mean_b * s_b
        for p in range(Q):
            wap = wt_ref[64 * p:64 * (p + 1), :]
            wbp = wt_ref[C + 64 * p:C + 64 * (p + 1), :]
            acc += _dot(xz_scr[:, p, :] * s_a[p:p + 1, :], wap)
            acc += _dot(o_scr[:, p, :] * s_b[p:p + 1, :], wbp)
            acc += _dot(shift_a[p:p + 1, :], wap)
            acc += _dot(shift_b[p:p + 1, :], wbp)
        vout_ref[...] = acc


def kernel(Di, DiA, v, f, g0, be0, W0, b0, g1, be1, W1, b1):
    Di2 = Di.reshape(Q * NF, Q * NN)
    DiA2 = DiA.reshape(Q * NN, Q * NF)
    v2 = v.reshape(NN, C)
    f2 = f.reshape(NF, C)
    xr_raw = v2.reshape(Q * NN, 64)
    W0t = W0.T
    W1t = W1.T
    g0a = g0[:C].reshape(1, C)
    g0b = g0[C:].reshape(Q, 64)
    be0a = be0[:C].reshape(1, C)
    be0b = be0[C:].reshape(Q, 64)
    g1a = g1[:C].reshape(Q, 64)
    g1b = g1[C:].reshape(Q, 64)
    be1a = be1[:C].reshape(Q, 64)
    be1b = be1[C:].reshape(Q, 64)
    b0r = b0.reshape(1, C)
    b1r = b1.reshape(1, C)

    zero = lambda i: (0, 0)

    xe, f_out, y = pl.pallas_call(
        _phase1_kernel,
        grid=(S1 + 1,),
        in_specs=[pl.BlockSpec((Q * NN, 64), zero),
                  pl.BlockSpec((BF, Q * NN),
                               lambda i: (jnp.minimum(i, S1 - 1), 0)),
                  pl.BlockSpec((BF // Q, C),
                               lambda i: (jnp.minimum(i, S1 - 1), 0)),
                  pl.BlockSpec((2 * C, C), zero),
                  pl.BlockSpec((1, C), zero),
                  pl.BlockSpec((Q, 64), zero),
                  pl.BlockSpec((1, C), zero),
                  pl.BlockSpec((Q, 64), zero),
                  pl.BlockSpec((1, C), zero)],
        out_specs=(pl.BlockSpec((Q * NN, 64), zero),
                   pl.BlockSpec((NF, C), zero),
                   pl.BlockSpec((NF, C), zero)),
        out_shape=(jax.ShapeDtypeStruct((Q * NN, 64), jnp.float32),
                   jax.ShapeDtypeStruct((NF, C), jnp.float32),
                   jax.ShapeDtypeStruct((NF, C), jnp.float32)),
        scratch_shapes=[pltpu.VMEM((Q * NN, 64), jnp.float32),
                        pltpu.VMEM((NF, C), jnp.float32),
                        pltpu.VMEM((NF, Q, 64), jnp.float32),
                        pltpu.VMEM((1, C), jnp.float32),
                        pltpu.VMEM((1, C), jnp.float32),
                        pltpu.VMEM((Q, 64), jnp.float32),
                        pltpu.VMEM((Q, 64), jnp.float32)],
        compiler_params=pltpu.CompilerParams(
            dimension_semantics=("arbitrary",)),
    )(xr_raw, Di2, f2, W0t, g0a, g0b, be0a, be0b, b0r)

    yr = y.reshape(Q * NF, 64)

    v_out = pl.pallas_call(
        _phase2_kernel,
        grid=(S2 + 1,),
        in_specs=[pl.BlockSpec((Q * NF, 64), zero),
                  pl.BlockSpec((BD, Q * NF),
                               lambda i: (jnp.minimum(i, S2 - 1), 0)),
                  pl.BlockSpec((Q * NN, 64), zero),
                  pl.BlockSpec((NN, C), zero),
                  pl.BlockSpec((2 * C, C), zero),
                  pl.BlockSpec((Q, 64), zero),
                  pl.BlockSpec((Q, 64), zero),
                  pl.BlockSpec((Q, 64), zero),
                  pl.BlockSpec((Q, 64), zero),
                  pl.BlockSpec((1, C), zero)],
        out_specs=pl.BlockSpec((NN, C), zero),
        out_shape=jax.ShapeDtypeStruct((NN, C), jnp.float32),
        scratch_shapes=[pltpu.VMEM((NN, Q, 64), jnp.float32),
                        pltpu.VMEM((NN, Q, 64), jnp.float32),
                        pltpu.VMEM((Q, 64), jnp.float32),
                        pltpu.VMEM((Q, 64), jnp.float32)],
        compiler_params=pltpu.CompilerParams(
            dimension_semantics=("arbitrary",)),
    )(yr, DiA2, xe, v2, W1t, g1a, g1b, be1a, be1b, b1r)

    return (v_out.reshape(v.shape), f_out.reshape(f.shape))


# speedup vs baseline: 1.0874x; 1.0103x over previous
import jax
import jax.numpy as jnp
from jax.experimental import pallas as pl
from jax.experimental.pallas import tpu as pltpu

C = 256
Q = 4
NF = 2048
NN = 1024
BF = 512
BD = 512
S1 = 16
S2 = 8


def _elu(x):
    return jnp.where(x > 0, x, jnp.exp(x) - 1.0)


def _dot(a, b):
    return jax.lax.dot_general(
        a, b, (((1,), (0,)), ((), ())),
        precision=jax.lax.Precision.DEFAULT,
        preferred_element_type=jnp.float32)


def _phase1_kernel(xr_ref, di_ref, f_ref, wt_ref, g0a_ref, g0b_ref,
                   be0a_ref, be0b_ref, b0_ref,
                   xe_ref, fout_ref, y_ref,
                   xe_scr, o_scr, om_s, om_ss):
    i = pl.program_id(0)

    @pl.when(i == 0)
    def _():
        xe = _elu(xr_ref[...])
        xe_scr[...] = xe
        xe_ref[...] = xe

    @pl.when(i < S1)
    def _():
        o = _dot(di_ref[...], xe_scr[...])
        o3 = o.reshape(BF // Q, Q, 64)
        o_scr[pl.ds(i * (BF // Q), BF // Q)] = o3

        @pl.when(i == 0)
        def _():
            om_s[...] = jnp.sum(o3, axis=0)
            om_ss[...] = jnp.sum(o3 * o3, axis=0)

        @pl.when(i > 0)
        def _():
            om_s[...] += jnp.sum(o3, axis=0)
            om_ss[...] += jnp.sum(o3 * o3, axis=0)

    @pl.when(i == S1)
    def _():
        inv_n = 1.0 / NF
        fin = _elu(f_ref[...])
        mean_a = jnp.sum(fin, axis=0, keepdims=True) * inv_n
        var_a = (jnp.sum(fin * fin, axis=0, keepdims=True) * inv_n
                 - mean_a * mean_a)
        s_a = g0a_ref[...] * jax.lax.rsqrt(var_a + 1e-5)
        mean_b = om_s[...] * inv_n
        var_b = om_ss[...] * inv_n - mean_b * mean_b
        s_b = g0b_ref[...] * jax.lax.rsqrt(var_b + 1e-5)

        wa = wt_ref[:C, :]
        acc = _dot(fin * s_a, wa)
        cvec = b0_ref[...] + _dot(be0a_ref[...] - mean_a * s_a, wa)
        shift_b = be0b_ref[...] - mean_b * s_b
        for p in range(Q):
            wp = wt_ref[C + 64 * p:C + 64 * (p + 1), :]
            acc += _dot(o_scr[:, p, :] * s_b[p:p + 1, :], wp)
            cvec += _dot(shift_b[p:p + 1, :], wp)
        fo = acc + cvec
        fout_ref[...] = fo
        y_ref[...] = _elu(fo)


def _phase2_kernel(y_ref, da_ref, xe_ref, v_ref, wt_ref, g1a_ref, g1b_ref,
                   be1a_ref, be1b_ref, b1_ref,
                   vout_ref, xz_scr, o_scr, om_s, om_ss):
    i = pl.program_id(0)

    @pl.when(i == 0)
    def _():
        xz_scr[...] = xe_ref[...].reshape(NN, Q, 64)

    @pl.when(i < S2)
    def _():
        o = _dot(da_ref[...], y_ref[...])
        o3 = o.reshape(BD // Q, Q, 64)
        o_scr[pl.ds(i * (BD // Q), BD // Q)] = o3

        @pl.when(i == 0)
        def _():
            om_s[...] = jnp.sum(o3, axis=0)
            om_ss[...] = jnp.sum(o3 * o3, axis=0)

        @pl.when(i > 0)
        def _():
            om_s[...] += jnp.sum(o3, axis=0)
            om_ss[...] += jnp.sum(o3 * o3, axis=0)

    @pl.when(i == S2)
    def _():
        inv_n = 1.0 / NN
        xz = xz_scr[...]
        xm = jnp.sum(xz, axis=0) * inv_n
        xv = jnp.sum(xz * xz, axis=0) * inv_n - xm * xm
        s_a = g1a_ref[...] * jax.lax.rsqrt(xv + 1e-5)
        mean_b = om_s[...] * inv_n
        var_b = om_ss[...] * inv_n - mean_b * mean_b
        s_b = g1b_ref[...] * jax.lax.rsqrt(var_b + 1e-5)

        acc = v_ref[...] + b1_ref[...]
        shift_a = be1a_ref[...] - xm * s_a
        shift_b = be1b_ref[...] - mean_b * s_b
        for p in range(Q):
            wap = wt_ref[64 * p:64 * (p + 1), :]
            wbp = wt_ref[C + 64 * p:C + 64 * (p + 1), :]
            acc += _dot(xz_scr[:, p, :] * s_a[p:p + 1, :], wap)
            acc += _dot(o_scr[:, p, :] * s_b[p:p + 1, :], wbp)
            acc += _dot(shift_a[p:p + 1, :], wap)
            acc += _dot(shift_b[p:p + 1, :], wbp)
        vout_ref[...] = acc


def kernel(Di, DiA, v, f, g0, be0, W0, b0, g1, be1, W1, b1):
    Di2 = Di.reshape(Q * NF, Q * NN)
    DiA2 = DiA.reshape(Q * NN, Q * NF)
    v2 = v.reshape(NN, C)
    f2 = f.reshape(NF, C)
    xr_raw = v2.reshape(Q * NN, 64)
    W0t = W0.T
    W1t = W1.T
    g0a = g0[:C].reshape(1, C)
    g0b = g0[C:].reshape(Q, 64)
    be0a = be0[:C].reshape(1, C)
    be0b = be0[C:].reshape(Q, 64)
    g1a = g1[:C].reshape(Q, 64)
    g1b = g1[C:].reshape(Q, 64)
    be1a = be1[:C].reshape(Q, 64)
    be1b = be1[C:].reshape(Q, 64)
    b0r = b0.reshape(1, C)
    b1r = b1.reshape(1, C)

    zero = lambda i: (0, 0)

    xe, f_out, y = pl.pallas_call(
        _phase1_kernel,
        grid=(S1 + 1,),
        in_specs=[pl.BlockSpec((Q * NN, 64), zero),
                  pl.BlockSpec((BF, Q * NN),
                               lambda i: (jnp.minimum(i, S1 - 1), 0)),
                  pl.BlockSpec((NF, C), zero),
                  pl.BlockSpec((2 * C, C), zero),
                  pl.BlockSpec((1, C), zero),
                  pl.BlockSpec((Q, 64), zero),
                  pl.BlockSpec((1, C), zero),
                  pl.BlockSpec((Q, 64), zero),
                  pl.BlockSpec((1, C), zero)],
        out_specs=(pl.BlockSpec((Q * NN, 64), zero),
                   pl.BlockSpec((NF, C), zero),
                   pl.BlockSpec((NF, C), zero)),
        out_shape=(jax.ShapeDtypeStruct((Q * NN, 64), jnp.float32),
                   jax.ShapeDtypeStruct((NF, C), jnp.float32),
                   jax.ShapeDtypeStruct((NF, C), jnp.float32)),
        scratch_shapes=[pltpu.VMEM((Q * NN, 64), jnp.float32),
                        pltpu.VMEM((NF, Q, 64), jnp.float32),
                        pltpu.VMEM((Q, 64), jnp.float32),
                        pltpu.VMEM((Q, 64), jnp.float32)],
        compiler_params=pltpu.CompilerParams(
            dimension_semantics=("arbitrary",)),
    )(xr_raw, Di2, f2, W0t, g0a, g0b, be0a, be0b, b0r)

    yr = y.reshape(Q * NF, 64)

    v_out = pl.pallas_call(
        _phase2_kernel,
        grid=(S2 + 1,),
        in_specs=[pl.BlockSpec((Q * NF, 64), zero),
                  pl.BlockSpec((BD, Q * NF),
                               lambda i: (jnp.minimum(i, S2 - 1), 0)),
                  pl.BlockSpec((Q * NN, 64), zero),
                  pl.BlockSpec((NN, C), zero),
                  pl.BlockSpec((2 * C, C), zero),
                  pl.BlockSpec((Q, 64), zero),
                  pl.BlockSpec((Q, 64), zero),
                  pl.BlockSpec((Q, 64), zero),
                  pl.BlockSpec((Q, 64), zero),
                  pl.BlockSpec((1, C), zero)],
        out_specs=pl.BlockSpec((NN, C), zero),
        out_shape=jax.ShapeDtypeStruct((NN, C), jnp.float32),
        scratch_shapes=[pltpu.VMEM((NN, Q, 64), jnp.float32),
                        pltpu.VMEM((NN, Q, 64), jnp.float32),
                        pltpu.VMEM((Q, 64), jnp.float32),
                        pltpu.VMEM((Q, 64), jnp.float32)],
        compiler_params=pltpu.CompilerParams(
            dimension_semantics=("arbitrary",)),
    )(yr, DiA2, xe, v2, W1t, g1a, g1b, be1a, be1b, b1r)

    return (v_out.reshape(v.shape), f_out.reshape(f.shape))
